# baseline (speedup 1.0000x reference)
import functools

import jax
import jax.numpy as jnp
from jax import lax
from jax.experimental import pallas as pl
from jax.experimental.pallas import tpu as pltpu
from jax.experimental.pallas import tpu_sc as plsc

NC, NS, LANES = 2, 16, 16
NW = NC * NS

BATCH, SEQ, DIM = 4, 8192, 1024
ROWS_PER_W = SEQ // NW
R = 8
NBLK = ROWS_PER_W // R
DEPTH = 3
UNROLL = 16
NB_MAIN = (NBLK // DEPTH) * DEPTH


def _sc_body(x_hbm, w_hbm, o_hbm, *refs):
    xbufs = refs[0:DEPTH]
    wq = refs[DEPTH:2 * DEPTH]
    sems = refs[2 * DEPTH:]
    sins = sems[0:DEPTH]
    souts = sems[DEPTH:2 * DEPTH]
    swq = sems[2 * DEPTH:]

    wid = lax.axis_index("s") * NC + lax.axis_index("c")
    row0 = wid * ROWS_PER_W

    def wslice(blk):
        return w_hbm.at[pl.ds(row0 + blk * R, R)]

    def xslice(ref, blk):
        return ref.at[:, pl.ds(row0 + blk * R, R), :]

    def add_block(wbuf, xbuf):
        @plsc.parallel_loop(0, BATCH)
        def _batches(b):
            @plsc.parallel_loop(0, R)
            def _rows(r):
                @plsc.parallel_loop(0, DIM, step=LANES, unroll=UNROLL)
                def _cols(c):
                    wv = wbuf[r, pl.ds(c, LANES)]
                    plsc.addupdate(xbuf.at[b, r, pl.ds(c, LANES)], wv)

    def half(blk, q, tail=False):
        qn = (q + 2) % DEPTH
        pltpu.make_async_copy(wslice(blk), wq[q], swq[q]).wait()
        pltpu.make_async_copy(xslice(x_hbm, blk), xbufs[q], sins[q]).wait()
        add_block(wq[q], xbufs[q])
        pltpu.async_copy(xbufs[q], xslice(o_hbm, blk), souts[q])

        if tail:
            return

        @pl.when(blk + 2 < NBLK)
        def _prep():
            pltpu.async_copy(wslice(blk + 2), wq[qn], swq[qn])

            @pl.when(blk > 0)
            def _drain():
                pltpu.make_async_copy(xbufs[qn], xslice(o_hbm, blk - 1),
                                      souts[qn]).wait()
            pltpu.async_copy(xslice(x_hbm, blk + 2), xbufs[qn], sins[qn])

    for blk in (0, 1):
        pltpu.async_copy(wslice(blk), wq[blk], swq[blk])
        pltpu.async_copy(xslice(x_hbm, blk), xbufs[blk], sins[blk])

    def body(i, _):
        blk = DEPTH * i
        half(blk, 0)
        half(blk + 1, 1)
        half(blk + 2, 2)
        return _

    lax.fori_loop(0, NB_MAIN // DEPTH, body, 0)
    for blk in range(NB_MAIN, NBLK):
        half(blk, blk % DEPTH, tail=True)
    for blk in range(NBLK - DEPTH, NBLK):
        pltpu.make_async_copy(xbufs[blk % DEPTH], xslice(o_hbm, blk),
                              souts[blk % DEPTH]).wait()


@functools.partial(jax.jit, static_argnums=())
def kernel(inputs, W):
    batch, seq_len, dim = inputs.shape
    run = pl.kernel(
        _sc_body,
        out_type=jax.ShapeDtypeStruct((batch, seq_len, dim), inputs.dtype),
        mesh=plsc.VectorSubcoreMesh(core_axis_name="c", subcore_axis_name="s"),
        compiler_params=pltpu.CompilerParams(use_tc_tiling_on_sc=True),
        scratch_types=(
            [pltpu.VMEM((BATCH, R, DIM), jnp.float32)] * DEPTH
            + [pltpu.VMEM((R, DIM), jnp.float32)] * DEPTH
            + [pltpu.SemaphoreType.DMA] * (3 * DEPTH)
        ),
    )
    return run(inputs, W)
